# Initial kernel scaffold; baseline (speedup 1.0000x reference)
import jax
import jax.numpy as jnp
from jax.experimental import pallas as pl


def kernel(x, table, W1, b1, W2, b2, W3, b3):
    raise NotImplementedError("write your pallas kernel here")


# speedup vs baseline: 16.9171x; 16.9171x over previous
import functools

import jax
import jax.numpy as jnp
from jax import lax
from jax.experimental import pallas as pl
from jax.experimental.pallas import tpu as pltpu
from jax.experimental.pallas import tpu_sc as plsc

N = 26
EMB = 32
OUT = 64
H1 = 256
H2 = 128
BATCH = 16384

NC = 2
NS = 16
NW = NC * NS

B_FLAT = BATCH * N
ROWS_PER_W = B_FLAT // NW
IDX_CHUNK = 128
IDX_ROWS = ROWS_PER_W // IDX_CHUNK
GATHERS_PER_STEP = 8
STEPS = IDX_ROWS // GATHERS_PER_STEP
STEP_ROWS = GATHERS_PER_STEP * IDX_CHUNK


def _sc_gather(table, idx):
    mesh = plsc.VectorSubcoreMesh(
        core_axis_name="c", subcore_axis_name="s", num_cores=NC,
        num_subcores=NS)

    @functools.partial(
        pl.kernel,
        out_type=jax.ShapeDtypeStruct((B_FLAT, EMB), jnp.float32),
        mesh=mesh,
        compiler_params=pltpu.CompilerParams(use_tc_tiling_on_sc=False),
        scratch_types=[
            pltpu.VMEM((IDX_ROWS, IDX_CHUNK), jnp.int32),
            pltpu.VMEM((STEP_ROWS, EMB), jnp.float32),
            pltpu.SemaphoreType.DMA,
        ],
    )
    def k(table_hbm, idx_hbm, out_hbm, idx_v, rows_v, sem):
        wid = lax.axis_index("s") * NC + lax.axis_index("c")
        base = wid * ROWS_PER_W
        pltpu.sync_copy(idx_hbm.at[wid], idx_v)

        def step(i, carry):
            handles = []
            for j in range(GATHERS_PER_STEP):
                handles.append(pltpu.async_copy(
                    table_hbm.at[idx_v.at[i * GATHERS_PER_STEP + j]],
                    rows_v.at[pl.ds(j * IDX_CHUNK, IDX_CHUNK)],
                    sem))
            for h in handles:
                h.wait()
            pltpu.sync_copy(rows_v, out_hbm.at[pl.ds(base + i * STEP_ROWS,
                                                     STEP_ROWS)])
            return carry

        lax.fori_loop(0, STEPS, step, 0)

    return k(table, idx)


def _mlp_body(h_ref, w1_ref, b1_ref, w2_ref, b2_ref, w3_ref, b3_ref, o_ref):
    h = h_ref[...]
    z = jnp.dot(h, w1_ref[...], preferred_element_type=jnp.float32)
    z = jnp.maximum(z + b1_ref[...], 0.0)
    z = jnp.dot(z, w2_ref[...], preferred_element_type=jnp.float32)
    z = jnp.maximum(z + b2_ref[...], 0.0)
    z = jnp.dot(z, w3_ref[...], preferred_element_type=jnp.float32)
    o_ref[...] = z + b3_ref[...]


def _mlp(h, W1, b1, W2, b2, W3, b3, block_b=1024):
    d_in = h.shape[1]
    grid = (BATCH // block_b,)
    return pl.pallas_call(
        _mlp_body,
        grid=grid,
        in_specs=[
            pl.BlockSpec((block_b, d_in), lambda i: (i, 0)),
            pl.BlockSpec((d_in, H1), lambda i: (0, 0)),
            pl.BlockSpec((1, H1), lambda i: (0, 0)),
            pl.BlockSpec((H1, H2), lambda i: (0, 0)),
            pl.BlockSpec((1, H2), lambda i: (0, 0)),
            pl.BlockSpec((H2, OUT), lambda i: (0, 0)),
            pl.BlockSpec((1, OUT), lambda i: (0, 0)),
        ],
        out_specs=pl.BlockSpec((block_b, OUT), lambda i: (i, 0)),
        out_shape=jax.ShapeDtypeStruct((BATCH, OUT), jnp.float32),
    )(h, W1, b1.reshape(1, H1), W2, b2.reshape(1, H2), W3,
      b3.reshape(1, OUT))


def kernel(x, table, W1, b1, W2, b2, W3, b3):
    idx = x.astype(jnp.int32).reshape(NW, IDX_ROWS, IDX_CHUNK)
    emb = _sc_gather(table, idx)
    h = emb.reshape(BATCH, N * EMB)
    return _mlp(h, W1, b1, W2, b2, W3, b3)


# speedup vs baseline: 17.0585x; 1.0084x over previous
import functools

import jax
import jax.numpy as jnp
from jax import lax
from jax.experimental import pallas as pl
from jax.experimental.pallas import tpu as pltpu
from jax.experimental.pallas import tpu_sc as plsc

N = 26
EMB = 32
OUT = 64
H1 = 256
H2 = 128
BATCH = 16384

NC = 2
NS = 16
NW = NC * NS

B_FLAT = BATCH * N
ROWS_PER_W = B_FLAT // NW
IDX_CHUNK = 128
IDX_ROWS = ROWS_PER_W // IDX_CHUNK
GATHERS_PER_STEP = 8
STEPS = IDX_ROWS // GATHERS_PER_STEP
STEP_ROWS = GATHERS_PER_STEP * IDX_CHUNK


def _sc_gather(table, idx):
    mesh = plsc.VectorSubcoreMesh(
        core_axis_name="c", subcore_axis_name="s", num_cores=NC,
        num_subcores=NS)

    @functools.partial(
        pl.kernel,
        out_type=jax.ShapeDtypeStruct((B_FLAT, EMB), jnp.float32),
        mesh=mesh,
        compiler_params=pltpu.CompilerParams(use_tc_tiling_on_sc=False),
        scratch_types=[
            pltpu.VMEM((IDX_ROWS, IDX_CHUNK), jnp.int32),
            pltpu.VMEM((2, STEP_ROWS, EMB), jnp.float32),
            pltpu.SemaphoreType.DMA,
            pltpu.SemaphoreType.DMA,
            pltpu.SemaphoreType.DMA,
            pltpu.SemaphoreType.DMA,
        ],
    )
    def k(table_hbm, idx_hbm, out_hbm, idx_v, rows_v, g0, g1, w0, w1):
        wid = lax.axis_index("s") * NC + lax.axis_index("c")
        base = wid * ROWS_PER_W
        gsem = (g0, g1)
        wsem = (w0, w1)
        pltpu.sync_copy(idx_hbm.at[wid], idx_v)

        def fire_gathers(step_idx, buf, sem):
            for j in range(GATHERS_PER_STEP):
                pltpu.async_copy(
                    table_hbm.at[idx_v.at[step_idx * GATHERS_PER_STEP + j]],
                    rows_v.at[buf, pl.ds(j * IDX_CHUNK, IDX_CHUNK)],
                    sem)

        def wait_bytes(buf, sem, out_off):
            pltpu.make_async_copy(
                rows_v.at[buf],
                out_hbm.at[pl.ds(out_off, STEP_ROWS)],
                sem).wait()

        fire_gathers(0, 0, gsem[0])
        for s in range(STEPS):
            buf = s % 2
            nxt = (s + 1) % 2
            if s + 1 < STEPS:
                if s >= 1:
                    wait_bytes(nxt, wsem[nxt], base)
                fire_gathers(s + 1, nxt, gsem[nxt])
            wait_bytes(buf, gsem[buf], base)
            pltpu.async_copy(rows_v.at[buf],
                             out_hbm.at[pl.ds(base + s * STEP_ROWS,
                                              STEP_ROWS)],
                             wsem[buf])
        wait_bytes(0, wsem[(STEPS - 1) % 2], base)
        wait_bytes(1, wsem[STEPS % 2], base)

    return k(table, idx)


def _mlp_body(h_ref, w1_ref, b1_ref, w2_ref, b2_ref, w3_ref, b3_ref, o_ref):
    h = h_ref[...]
    z = jnp.dot(h, w1_ref[...], preferred_element_type=jnp.float32)
    z = jnp.maximum(z + b1_ref[...], 0.0)
    z = jnp.dot(z, w2_ref[...], preferred_element_type=jnp.float32)
    z = jnp.maximum(z + b2_ref[...], 0.0)
    z = jnp.dot(z, w3_ref[...], preferred_element_type=jnp.float32)
    o_ref[...] = z + b3_ref[...]


def _mlp(h, W1, b1, W2, b2, W3, b3, block_b=1024):
    d_in = h.shape[1]
    grid = (BATCH // block_b,)
    return pl.pallas_call(
        _mlp_body,
        grid=grid,
        in_specs=[
            pl.BlockSpec((block_b, d_in), lambda i: (i, 0)),
            pl.BlockSpec((d_in, H1), lambda i: (0, 0)),
            pl.BlockSpec((1, H1), lambda i: (0, 0)),
            pl.BlockSpec((H1, H2), lambda i: (0, 0)),
            pl.BlockSpec((1, H2), lambda i: (0, 0)),
            pl.BlockSpec((H2, OUT), lambda i: (0, 0)),
            pl.BlockSpec((1, OUT), lambda i: (0, 0)),
        ],
        out_specs=pl.BlockSpec((block_b, OUT), lambda i: (i, 0)),
        out_shape=jax.ShapeDtypeStruct((BATCH, OUT), jnp.float32),
    )(h, W1, b1.reshape(1, H1), W2, b2.reshape(1, H2), W3,
      b3.reshape(1, OUT))


def kernel(x, table, W1, b1, W2, b2, W3, b3):
    idx = x.astype(jnp.int32).reshape(NW, IDX_ROWS, IDX_CHUNK)
    emb = _sc_gather(table, idx)
    h = emb.reshape(BATCH, N * EMB)
    return _mlp(h, W1, b1, W2, b2, W3, b3)
